# baseline (speedup 1.0000x reference)
import functools

import jax
import jax.numpy as jnp
from jax import lax
from jax.experimental import pallas as pl
from jax.experimental.pallas import tpu as pltpu
from jax.experimental.pallas import tpu_sc as plsc

N_TOTAL = 50000
E = 800000
DIM = 64
HALF = 32
K = 256
IDXB = 4
NBUF = 3
LOOK = 2
CPS = 200
NITER = CPS // IDXB
NBLKP = 16 * CPS
EP = NBLKP * K
NSUB = 16
RSUB = 3128
RLAST = N_TOTAL - 15 * RSUB

_f32 = jnp.float32
_i32 = jnp.int32
_DNUMS = lax.GatherDimensionNumbers(
    offset_dims=(), collapsed_slice_dims=(0,), start_index_map=(0,))


def _bcast_lane(vec16, jj):
    return lax.gather(vec16, jnp.full((16, 1), jj, _i32), _DNUMS,
                      slice_sizes=(1,),
                      mode=lax.GatherScatterMode.PROMISE_IN_BOUNDS)


def _sc_spmm(row2, col2, val2, ego3):
    mesh = plsc.VectorSubcoreMesh(core_axis_name="c", subcore_axis_name="s")

    @functools.partial(
        pl.kernel,
        out_type=jax.ShapeDtypeStruct((2, N_TOTAL, HALF), _f32),
        mesh=mesh,
        scratch_types=[
            pltpu.VMEM((2, IDXB, K), _i32),
            pltpu.VMEM((2, IDXB, K), _i32),
            pltpu.VMEM((2, IDXB, K), _f32),
            [pltpu.VMEM((K, HALF), _f32)] * NBUF,
            pltpu.VMEM_SHARED((N_TOTAL, HALF), _f32),
            [pltpu.SemaphoreType.DMA] * NBUF,
            [pltpu.SemaphoreType.DMA] * NBUF,
            pltpu.SemaphoreType.DMA,
        ],
        compiler_params=pltpu.CompilerParams(use_tc_tiling_on_sc=False),
    )
    def spmm(row_h, col_h, val_h, ego_h, side_h,
             rowv, colv, valv, bufs, side_sh, gsems, ssems, isem):
        cid = lax.axis_index("c")
        sid = lax.axis_index("s")

        zero16 = jnp.zeros((16,), _f32)

        def zrow(r, carry):
            bufs[0][r, pl.ds(0, 16)] = zero16
            bufs[0][r, pl.ds(16, 16)] = zero16
            return carry

        lax.fori_loop(0, K, zrow, 0)

        def zcopy(i, carry):
            pltpu.sync_copy(
                bufs[0], side_sh.at[pl.ds(sid * RSUB + i * K, K)])
            return carry

        lax.fori_loop(0, 12, zcopy, 0)

        @pl.when(sid < 15)
        def _():
            pltpu.sync_copy(bufs[0].at[pl.ds(0, 56)],
                            side_sh.at[pl.ds(sid * RSUB + 12 * K, 56)])

        @pl.when(sid == 15)
        def _():
            pltpu.sync_copy(bufs[0].at[pl.ds(0, 8)],
                            side_sh.at[pl.ds(15 * RSUB + 12 * K, 8)])

        plsc.subcore_barrier()

        def scale(buf, valrow):
            def g_body(g, carry):
                val16 = valrow[pl.ds(g * 16, 16)]
                for jj in range(16):
                    r = g * 16 + jj
                    bv = _bcast_lane(val16, jj)
                    buf[r, pl.ds(0, 16)] = buf[r, pl.ds(0, 16)] * bv
                    buf[r, pl.ds(16, 16)] = buf[r, pl.ds(16, 16)] * bv
                return carry

            lax.fori_loop(0, K // 16, g_body, 0)

        ego_c = ego_h.at[cid]
        base0 = sid * CPS

        pltpu.sync_copy(col_h.at[pl.ds(base0, IDXB)], colv.at[0])
        pltpu.sync_copy(row_h.at[pl.ds(base0, IDXB)], rowv.at[0])
        pltpu.sync_copy(val_h.at[pl.ds(base0, IDXB)], valv.at[0])

        def one_iter(t, carry):
            tp = lax.rem(t, 2)
            tn = 1 - tp
            nbase = base0 + (t + 1) * IDXB

            @pl.when(t < NITER - 1)
            def _():
                pltpu.async_copy(col_h.at[pl.ds(nbase, IDXB)], colv.at[tn], isem)
                pltpu.async_copy(row_h.at[pl.ds(nbase, IDXB)], rowv.at[tn], isem)
                pltpu.async_copy(val_h.at[pl.ds(nbase, IDXB)], valv.at[tn], isem)

            cv = colv.at[tp]
            rv = rowv.at[tp]
            gd = [None] * NBUF
            sd = [None] * NBUF
            for s in range(LOOK):
                gd[s] = pltpu.async_copy(ego_c.at[cv.at[s]], bufs[s], gsems[s])
            for s in range(IDXB):
                b = s % NBUF
                if s + LOOK < IDXB:
                    nb = (s + LOOK) % NBUF
                    if sd[nb] is not None:
                        sd[nb].wait()
                    gd[nb] = pltpu.async_copy(
                        ego_c.at[cv.at[s + LOOK]], bufs[nb], gsems[nb])
                gd[b].wait()
                scale(bufs[b], valv.at[tp].at[s])
                sd[b] = pltpu.async_copy(
                    bufs[b], side_sh.at[rv.at[s]], ssems[b], add=True)
            for b in range(NBUF):
                sd[b].wait()

            @pl.when(t < NITER - 1)
            def _():
                pltpu.make_async_copy(col_h.at[pl.ds(nbase, IDXB)],
                                      colv.at[tn], isem).wait()
                pltpu.make_async_copy(row_h.at[pl.ds(nbase, IDXB)],
                                      rowv.at[tn], isem).wait()
                pltpu.make_async_copy(val_h.at[pl.ds(nbase, IDXB)],
                                      valv.at[tn], isem).wait()

            return carry

        lax.fori_loop(0, NITER, one_iter, 0)
        plsc.subcore_barrier()

        sl = pl.ds(sid * RSUB, RSUB)
        sl_last = pl.ds(15 * RSUB, RLAST)

        @pl.when(sid < 15)
        def _():
            pltpu.sync_copy(side_sh.at[sl], side_h.at[cid].at[sl])

        @pl.when(sid == 15)
        def _():
            pltpu.sync_copy(side_sh.at[sl_last], side_h.at[cid].at[sl_last])

    return spmm(row2, col2, val2, ego3)


BLK = 5000


def _tc_body(s3_ref, e3_ref, wg_ref, bg_ref, wb_ref, bb_ref,
             onext_ref, onorm_ref):
    s = jnp.concatenate([s3_ref[0], s3_ref[1]], axis=1)
    eg = jnp.concatenate([e3_ref[0], e3_ref[1]], axis=1)
    sum_emb = jnp.dot(s, wg_ref[...], preferred_element_type=_f32) + bg_ref[...]
    bi_emb = (jnp.dot(eg * s, wb_ref[...], preferred_element_type=_f32)
              + bb_ref[...])
    e = sum_emb + bi_emb
    e = jnp.where(e >= 0, e, 0.2 * e)
    nr = jnp.sqrt(jnp.sum(e * e, axis=1, keepdims=True))
    d = e / jnp.maximum(nr, 1e-12)
    onext_ref[0] = e[:, :HALF]
    onext_ref[1] = e[:, HALF:]
    onorm_ref[...] = d


def _tc_dense(side3, ego3, w_gcn, b_gcn, w_bi, b_bi):
    grid = (N_TOTAL // BLK,)
    pair_spec = pl.BlockSpec((2, BLK, HALF), lambda i: (0, i, 0))
    w_spec = pl.BlockSpec((DIM, DIM), lambda i: (0, 0))
    b_spec = pl.BlockSpec((1, DIM), lambda i: (0, 0))
    return pl.pallas_call(
        _tc_body,
        grid=grid,
        in_specs=[pair_spec, pair_spec, w_spec, b_spec, w_spec, b_spec],
        out_specs=[pair_spec, pl.BlockSpec((BLK, DIM), lambda i: (i, 0))],
        out_shape=[
            jax.ShapeDtypeStruct((2, N_TOTAL, HALF), _f32),
            jax.ShapeDtypeStruct((N_TOTAL, DIM), _f32),
        ],
    )(side3, ego3, w_gcn, b_gcn, w_bi, b_bi)


def kernel(adj_indices, adj_values, user_embedding_weight, item_embedding_weight,
           W_gcn_0, b_gcn_0, W_bi_0, b_bi_0,
           W_gcn_1, b_gcn_1, W_bi_1, b_bi_1,
           W_gcn_2, b_gcn_2, W_bi_2, b_bi_2):
    padn = EP - E
    ar = jnp.arange(padn, dtype=_i32)
    prow = ar % N_TOTAL
    pcol = (ar * 61) % N_TOTAL
    row2 = jnp.concatenate([adj_indices[0].astype(_i32), prow]).reshape(NBLKP, K)
    col2 = jnp.concatenate([adj_indices[1].astype(_i32), pcol]).reshape(NBLKP, K)
    val2 = jnp.concatenate([adj_values, jnp.zeros((padn,), _f32)]).reshape(NBLKP, K)
    ego0 = jnp.concatenate([user_embedding_weight, item_embedding_weight], axis=0)
    ego3 = jnp.stack([ego0[:, :HALF], ego0[:, HALF:]])
    outs = [ego0]
    for (wg, bg, wb, bb) in ((W_gcn_0, b_gcn_0, W_bi_0, b_bi_0),
                             (W_gcn_1, b_gcn_1, W_bi_1, b_bi_1),
                             (W_gcn_2, b_gcn_2, W_bi_2, b_bi_2)):
        side3 = _sc_spmm(row2, col2, val2, ego3)
        ego3, nrm = _tc_dense(side3, ego3, wg, bg, wb, bb)
        outs.append(nrm)
    final = jnp.concatenate(outs, axis=1)
    n_users = user_embedding_weight.shape[0]
    return final[:n_users], final[n_users:]


# speedup vs baseline: 1.1051x; 1.1051x over previous
import functools

import jax
import jax.numpy as jnp
from jax import lax
from jax.experimental import pallas as pl
from jax.experimental.pallas import tpu as pltpu
from jax.experimental.pallas import tpu_sc as plsc

N_TOTAL = 50000
E = 800000
DIM = 64
HALF = 32
K = 128
IDXB = 16
NBUF = 4
LOOK = 2
CPS = 400
NITER = CPS // IDXB
NBLKP = 16 * CPS
EP = NBLKP * K
NSUB = 16
RSUB = 3128
RLAST = N_TOTAL - 15 * RSUB

_f32 = jnp.float32
_i32 = jnp.int32
_DNUMS = lax.GatherDimensionNumbers(
    offset_dims=(), collapsed_slice_dims=(0,), start_index_map=(0,))


def _bcast_lane(vec16, jj):
    return lax.gather(vec16, jnp.full((16, 1), jj, _i32), _DNUMS,
                      slice_sizes=(1,),
                      mode=lax.GatherScatterMode.PROMISE_IN_BOUNDS)


def _sc_spmm(row2, col2, val2, ego3):
    mesh = plsc.VectorSubcoreMesh(core_axis_name="c", subcore_axis_name="s")

    @functools.partial(
        pl.kernel,
        out_type=jax.ShapeDtypeStruct((2, N_TOTAL, HALF), _f32),
        mesh=mesh,
        scratch_types=[
            pltpu.VMEM((2, IDXB, K), _i32),
            pltpu.VMEM((2, IDXB, K), _i32),
            pltpu.VMEM((2, IDXB, K), _f32),
            [pltpu.VMEM((K, HALF), _f32)] * NBUF,
            pltpu.VMEM_SHARED((N_TOTAL, HALF), _f32),
            [pltpu.SemaphoreType.DMA] * NBUF,
            [pltpu.SemaphoreType.DMA] * NBUF,
            pltpu.SemaphoreType.DMA,
        ],
        compiler_params=pltpu.CompilerParams(use_tc_tiling_on_sc=False),
    )
    def spmm(row_h, col_h, val_h, ego_h, side_h,
             rowv, colv, valv, bufs, side_sh, gsems, ssems, isem):
        cid = lax.axis_index("c")
        sid = lax.axis_index("s")

        zero16 = jnp.zeros((16,), _f32)

        def zrow(r, carry):
            bufs[0][r, pl.ds(0, 16)] = zero16
            bufs[0][r, pl.ds(16, 16)] = zero16
            return carry

        lax.fori_loop(0, K, zrow, 0)

        def zcopy(i, carry):
            pltpu.sync_copy(
                bufs[0], side_sh.at[pl.ds(sid * RSUB + i * K, K)])
            return carry

        lax.fori_loop(0, 24, zcopy, 0)

        @pl.when(sid < 15)
        def _():
            pltpu.sync_copy(bufs[0].at[pl.ds(0, 56)],
                            side_sh.at[pl.ds(sid * RSUB + 24 * K, 56)])

        @pl.when(sid == 15)
        def _():
            pltpu.sync_copy(bufs[0].at[pl.ds(0, 8)],
                            side_sh.at[pl.ds(15 * RSUB + 24 * K, 8)])

        plsc.subcore_barrier()

        def scale(buf, valrow):
            def g_body(g, carry):
                val16 = valrow[pl.ds(g * 16, 16)]
                for jj in range(16):
                    r = g * 16 + jj
                    bv = _bcast_lane(val16, jj)
                    buf[r, pl.ds(0, 16)] = buf[r, pl.ds(0, 16)] * bv
                    buf[r, pl.ds(16, 16)] = buf[r, pl.ds(16, 16)] * bv
                return carry

            lax.fori_loop(0, K // 16, g_body, 0)

        ego_c = ego_h.at[cid]
        base0 = sid * CPS

        pltpu.sync_copy(col_h.at[pl.ds(base0, IDXB)], colv.at[0])
        pltpu.sync_copy(row_h.at[pl.ds(base0, IDXB)], rowv.at[0])
        pltpu.sync_copy(val_h.at[pl.ds(base0, IDXB)], valv.at[0])

        def one_iter(t, carry):
            tp = lax.rem(t, 2)
            tn = 1 - tp
            nbase = base0 + (t + 1) * IDXB

            @pl.when(t < NITER - 1)
            def _():
                pltpu.async_copy(col_h.at[pl.ds(nbase, IDXB)], colv.at[tn], isem)
                pltpu.async_copy(row_h.at[pl.ds(nbase, IDXB)], rowv.at[tn], isem)
                pltpu.async_copy(val_h.at[pl.ds(nbase, IDXB)], valv.at[tn], isem)

            cv = colv.at[tp]
            rv = rowv.at[tp]
            gd = [None] * NBUF
            sd = [None] * NBUF
            for s in range(LOOK):
                gd[s] = pltpu.async_copy(ego_c.at[cv.at[s]], bufs[s], gsems[s])
            for s in range(IDXB):
                b = s % NBUF
                if s + LOOK < IDXB:
                    nb = (s + LOOK) % NBUF
                    if sd[nb] is not None:
                        sd[nb].wait()
                    gd[nb] = pltpu.async_copy(
                        ego_c.at[cv.at[s + LOOK]], bufs[nb], gsems[nb])
                gd[b].wait()
                scale(bufs[b], valv.at[tp].at[s])
                sd[b] = pltpu.async_copy(
                    bufs[b], side_sh.at[rv.at[s]], ssems[b], add=True)
            for b in range(NBUF):
                sd[b].wait()

            @pl.when(t < NITER - 1)
            def _():
                pltpu.make_async_copy(col_h.at[pl.ds(nbase, IDXB)],
                                      colv.at[tn], isem).wait()
                pltpu.make_async_copy(row_h.at[pl.ds(nbase, IDXB)],
                                      rowv.at[tn], isem).wait()
                pltpu.make_async_copy(val_h.at[pl.ds(nbase, IDXB)],
                                      valv.at[tn], isem).wait()

            return carry

        lax.fori_loop(0, NITER, one_iter, 0)
        plsc.subcore_barrier()

        sl = pl.ds(sid * RSUB, RSUB)
        sl_last = pl.ds(15 * RSUB, RLAST)

        @pl.when(sid < 15)
        def _():
            pltpu.sync_copy(side_sh.at[sl], side_h.at[cid].at[sl])

        @pl.when(sid == 15)
        def _():
            pltpu.sync_copy(side_sh.at[sl_last], side_h.at[cid].at[sl_last])

    return spmm(row2, col2, val2, ego3)


BLK = 5000


def _tc_body(s3_ref, e3_ref, wg_ref, bg_ref, wb_ref, bb_ref,
             onext_ref, onorm_ref):
    s = jnp.concatenate([s3_ref[0], s3_ref[1]], axis=1)
    eg = jnp.concatenate([e3_ref[0], e3_ref[1]], axis=1)
    sum_emb = jnp.dot(s, wg_ref[...], preferred_element_type=_f32) + bg_ref[...]
    bi_emb = (jnp.dot(eg * s, wb_ref[...], preferred_element_type=_f32)
              + bb_ref[...])
    e = sum_emb + bi_emb
    e = jnp.where(e >= 0, e, 0.2 * e)
    nr = jnp.sqrt(jnp.sum(e * e, axis=1, keepdims=True))
    d = e / jnp.maximum(nr, 1e-12)
    onext_ref[0] = e[:, :HALF]
    onext_ref[1] = e[:, HALF:]
    onorm_ref[...] = d


def _tc_dense(side3, ego3, w_gcn, b_gcn, w_bi, b_bi):
    grid = (N_TOTAL // BLK,)
    pair_spec = pl.BlockSpec((2, BLK, HALF), lambda i: (0, i, 0))
    w_spec = pl.BlockSpec((DIM, DIM), lambda i: (0, 0))
    b_spec = pl.BlockSpec((1, DIM), lambda i: (0, 0))
    return pl.pallas_call(
        _tc_body,
        grid=grid,
        in_specs=[pair_spec, pair_spec, w_spec, b_spec, w_spec, b_spec],
        out_specs=[pair_spec, pl.BlockSpec((BLK, DIM), lambda i: (i, 0))],
        out_shape=[
            jax.ShapeDtypeStruct((2, N_TOTAL, HALF), _f32),
            jax.ShapeDtypeStruct((N_TOTAL, DIM), _f32),
        ],
    )(side3, ego3, w_gcn, b_gcn, w_bi, b_bi)


def kernel(adj_indices, adj_values, user_embedding_weight, item_embedding_weight,
           W_gcn_0, b_gcn_0, W_bi_0, b_bi_0,
           W_gcn_1, b_gcn_1, W_bi_1, b_bi_1,
           W_gcn_2, b_gcn_2, W_bi_2, b_bi_2):
    padn = EP - E
    ar = jnp.arange(padn, dtype=_i32)
    prow = ar % N_TOTAL
    pcol = (ar * 61) % N_TOTAL
    row2 = jnp.concatenate([adj_indices[0].astype(_i32), prow]).reshape(NBLKP, K)
    col2 = jnp.concatenate([adj_indices[1].astype(_i32), pcol]).reshape(NBLKP, K)
    val2 = jnp.concatenate([adj_values, jnp.zeros((padn,), _f32)]).reshape(NBLKP, K)
    ego0 = jnp.concatenate([user_embedding_weight, item_embedding_weight], axis=0)
    ego3 = jnp.stack([ego0[:, :HALF], ego0[:, HALF:]])
    outs = [ego0]
    for (wg, bg, wb, bb) in ((W_gcn_0, b_gcn_0, W_bi_0, b_bi_0),
                             (W_gcn_1, b_gcn_1, W_bi_1, b_bi_1),
                             (W_gcn_2, b_gcn_2, W_bi_2, b_bi_2)):
        side3 = _sc_spmm(row2, col2, val2, ego3)
        ego3, nrm = _tc_dense(side3, ego3, wg, bg, wb, bb)
        outs.append(nrm)
    final = jnp.concatenate(outs, axis=1)
    n_users = user_embedding_weight.shape[0]
    return final[:n_users], final[n_users:]


# speedup vs baseline: 1.1566x; 1.0466x over previous
import functools

import jax
import jax.numpy as jnp
from jax import lax
from jax.experimental import pallas as pl
from jax.experimental.pallas import tpu as pltpu
from jax.experimental.pallas import tpu_sc as plsc

N_TOTAL = 50000
E = 800000
DIM = 64
HALF = 32
K = 128
IDXB = 16
NBUF = 4
LOOK = 2
CPS = 400
NITER = CPS // IDXB
NBLKP = 16 * CPS
EP = NBLKP * K
NSUB = 16
RSUB = 3128
RLAST = N_TOTAL - 15 * RSUB

_f32 = jnp.float32
_i32 = jnp.int32
_DNUMS = lax.GatherDimensionNumbers(
    offset_dims=(), collapsed_slice_dims=(0,), start_index_map=(0,))


def _bcast_lane(vec16, jj):
    return lax.gather(vec16, jnp.full((16, 1), jj, _i32), _DNUMS,
                      slice_sizes=(1,),
                      mode=lax.GatherScatterMode.PROMISE_IN_BOUNDS)


def _sc_spmm(row2, col2, val2, ego3):
    mesh = plsc.VectorSubcoreMesh(core_axis_name="c", subcore_axis_name="s")

    @functools.partial(
        pl.kernel,
        out_type=jax.ShapeDtypeStruct((2, N_TOTAL, HALF), _f32),
        mesh=mesh,
        scratch_types=[
            pltpu.VMEM((2, IDXB, K), _i32),
            pltpu.VMEM((2, IDXB, K), _i32),
            pltpu.VMEM((2, IDXB, K), _f32),
            [pltpu.VMEM((K, HALF), _f32)] * NBUF,
            pltpu.VMEM_SHARED((N_TOTAL, HALF), _f32),
            [pltpu.SemaphoreType.DMA] * NBUF,
            [pltpu.SemaphoreType.DMA] * NBUF,
            pltpu.SemaphoreType.DMA,
        ],
        compiler_params=pltpu.CompilerParams(use_tc_tiling_on_sc=False),
    )
    def spmm(row_h, col_h, val_h, ego_h, side_h,
             rowv, colv, valv, bufs, side_sh, gsems, ssems, isem):
        cid = lax.axis_index("c")
        sid = lax.axis_index("s")

        zero16 = jnp.zeros((16,), _f32)

        def zrow(r, carry):
            bufs[0][r, pl.ds(0, 16)] = zero16
            bufs[0][r, pl.ds(16, 16)] = zero16
            return carry

        lax.fori_loop(0, K, zrow, 0)

        def zcopy(i, carry):
            pltpu.sync_copy(
                bufs[0], side_sh.at[pl.ds(sid * RSUB + i * K, K)])
            return carry

        lax.fori_loop(0, 24, zcopy, 0)

        @pl.when(sid < 15)
        def _():
            pltpu.sync_copy(bufs[0].at[pl.ds(0, 56)],
                            side_sh.at[pl.ds(sid * RSUB + 24 * K, 56)])

        @pl.when(sid == 15)
        def _():
            pltpu.sync_copy(bufs[0].at[pl.ds(0, 8)],
                            side_sh.at[pl.ds(15 * RSUB + 24 * K, 8)])

        plsc.subcore_barrier()

        def scale(buf, valrow):
            def g_body(g, carry):
                val16 = valrow[pl.ds(g * 16, 16)]
                for jj in range(16):
                    r = g * 16 + jj
                    bv = _bcast_lane(val16, jj)
                    buf[r, pl.ds(0, 16)] = buf[r, pl.ds(0, 16)] * bv
                    buf[r, pl.ds(16, 16)] = buf[r, pl.ds(16, 16)] * bv
                return carry

            lax.fori_loop(0, K // 16, g_body, 0)

        ego_c = ego_h.at[cid]
        base0 = sid * CPS

        pltpu.sync_copy(col_h.at[pl.ds(base0, IDXB)], colv.at[0])
        pltpu.sync_copy(row_h.at[pl.ds(base0, IDXB)], rowv.at[0])
        pltpu.sync_copy(val_h.at[pl.ds(base0, IDXB)], valv.at[0])

        def one_iter(t, carry):
            tp = lax.rem(t, 2)
            tn = 1 - tp
            nbase = base0 + (t + 1) * IDXB

            @pl.when(t < NITER - 1)
            def _():
                pltpu.async_copy(col_h.at[pl.ds(nbase, IDXB)], colv.at[tn], isem)
                pltpu.async_copy(row_h.at[pl.ds(nbase, IDXB)], rowv.at[tn], isem)
                pltpu.async_copy(val_h.at[pl.ds(nbase, IDXB)], valv.at[tn], isem)

            cv = colv.at[tp]
            rv = rowv.at[tp]
            gd = [None] * NBUF
            sd = [None] * NBUF
            for s in range(LOOK):
                gd[s] = pltpu.async_copy(ego_c.at[cv.at[s]], bufs[s], gsems[s])
            for s in range(IDXB):
                b = s % NBUF
                if s + LOOK < IDXB:
                    nb = (s + LOOK) % NBUF
                    if sd[nb] is not None:
                        sd[nb].wait()
                    gd[nb] = pltpu.async_copy(
                        ego_c.at[cv.at[s + LOOK]], bufs[nb], gsems[nb])
                gd[b].wait()
                scale(bufs[b], valv.at[tp].at[s])
                sd[b] = pltpu.async_copy(
                    bufs[b], side_sh.at[rv.at[s]], ssems[b], add=True)
            for b in range(NBUF):
                sd[b].wait()

            @pl.when(t < NITER - 1)
            def _():
                pltpu.make_async_copy(col_h.at[pl.ds(nbase, IDXB)],
                                      colv.at[tn], isem).wait()
                pltpu.make_async_copy(row_h.at[pl.ds(nbase, IDXB)],
                                      rowv.at[tn], isem).wait()
                pltpu.make_async_copy(val_h.at[pl.ds(nbase, IDXB)],
                                      valv.at[tn], isem).wait()

            return carry

        lax.fori_loop(0, NITER, one_iter, 0)
        plsc.subcore_barrier()

        sl = pl.ds(sid * RSUB, RSUB)
        sl_last = pl.ds(15 * RSUB, RLAST)

        @pl.when(sid < 15)
        def _():
            pltpu.sync_copy(side_sh.at[sl], side_h.at[cid].at[sl])

        @pl.when(sid == 15)
        def _():
            pltpu.sync_copy(side_sh.at[sl_last], side_h.at[cid].at[sl_last])

    return spmm(row2, col2, val2, ego3)


BLK = 5000


def _tc_body(s3_ref, e3_ref, wg_ref, bg_ref, wb_ref, bb_ref,
             onext_ref, onorm_ref):
    s = jnp.concatenate([s3_ref[0], s3_ref[1]], axis=1)
    eg = jnp.concatenate([e3_ref[0], e3_ref[1]], axis=1)
    sum_emb = jnp.dot(s, wg_ref[...], preferred_element_type=_f32) + bg_ref[...]
    bi_emb = (jnp.dot(eg * s, wb_ref[...], preferred_element_type=_f32)
              + bb_ref[...])
    e = sum_emb + bi_emb
    e = jnp.where(e >= 0, e, 0.2 * e)
    nr = jnp.sqrt(jnp.sum(e * e, axis=1, keepdims=True))
    d = e / jnp.maximum(nr, 1e-12)
    onext_ref[0] = e[:, :HALF]
    onext_ref[1] = e[:, HALF:]
    onorm_ref[...] = d


def _tc_dense(side3, ego3, w_gcn, b_gcn, w_bi, b_bi):
    grid = (N_TOTAL // BLK,)
    pair_spec = pl.BlockSpec((2, BLK, HALF), lambda i: (0, i, 0))
    w_spec = pl.BlockSpec((DIM, DIM), lambda i: (0, 0))
    b_spec = pl.BlockSpec((1, DIM), lambda i: (0, 0))
    return pl.pallas_call(
        _tc_body,
        grid=grid,
        in_specs=[pair_spec, pair_spec, w_spec, b_spec, w_spec, b_spec],
        out_specs=[pair_spec, pl.BlockSpec((BLK, DIM), lambda i: (i, 0))],
        out_shape=[
            jax.ShapeDtypeStruct((2, N_TOTAL, HALF), _f32),
            jax.ShapeDtypeStruct((N_TOTAL, DIM), _f32),
        ],
    )(side3, ego3, w_gcn, b_gcn, w_bi, b_bi)


PBLK = 5000


def _prep_body(u_ref, it_ref, out_ref):
    i = pl.program_id(0)

    @pl.when(i < 5)
    def _():
        e = u_ref[...]
        out_ref[0] = e[:, :HALF]
        out_ref[1] = e[:, HALF:]

    @pl.when(i >= 5)
    def _():
        e = it_ref[...]
        out_ref[0] = e[:, :HALF]
        out_ref[1] = e[:, HALF:]


def _prep_ego3(user_w, item_w):
    spec = pl.BlockSpec((PBLK, DIM), lambda i: (jnp.minimum(i, 4), 0))
    spec_it = pl.BlockSpec((PBLK, DIM), lambda i: (jnp.maximum(i - 5, 0), 0))
    return pl.pallas_call(
        _prep_body,
        grid=(10,),
        in_specs=[spec, spec_it],
        out_specs=pl.BlockSpec((2, PBLK, HALF), lambda i: (0, i, 0)),
        out_shape=jax.ShapeDtypeStruct((2, N_TOTAL, HALF), _f32),
    )(user_w, item_w)


def _asm_body(u_ref, it_ref, n1u_ref, n1i_ref, n2u_ref, n2i_ref,
              n3u_ref, n3i_ref, us_ref, is_ref):
    us_ref[...] = jnp.concatenate(
        [u_ref[...], n1u_ref[...], n2u_ref[...], n3u_ref[...]], axis=1)
    is_ref[...] = jnp.concatenate(
        [it_ref[...], n1i_ref[...], n2i_ref[...], n3i_ref[...]], axis=1)


def _assemble(user_w, item_w, n1, n2, n3):
    nu = N_TOTAL // 2
    spec_u = pl.BlockSpec((PBLK, DIM), lambda i: (i, 0))
    spec_i = pl.BlockSpec((PBLK, DIM), lambda i: (i + 5, 0))
    out_spec = pl.BlockSpec((PBLK, 4 * DIM), lambda i: (i, 0))
    return pl.pallas_call(
        _asm_body,
        grid=(5,),
        in_specs=[spec_u, spec_u, spec_u, spec_i, spec_u, spec_i,
                  spec_u, spec_i],
        out_specs=[out_spec, out_spec],
        out_shape=[jax.ShapeDtypeStruct((nu, 4 * DIM), _f32),
                   jax.ShapeDtypeStruct((nu, 4 * DIM), _f32)],
    )(user_w, item_w, n1, n1, n2, n2, n3, n3)


def kernel(adj_indices, adj_values, user_embedding_weight, item_embedding_weight,
           W_gcn_0, b_gcn_0, W_bi_0, b_bi_0,
           W_gcn_1, b_gcn_1, W_bi_1, b_bi_1,
           W_gcn_2, b_gcn_2, W_bi_2, b_bi_2):
    padn = EP - E
    ar = jnp.arange(padn, dtype=_i32)
    prow = ar % N_TOTAL
    pcol = (ar * 61) % N_TOTAL
    row2 = jnp.concatenate([adj_indices[0].astype(_i32), prow]).reshape(NBLKP, K)
    col2 = jnp.concatenate([adj_indices[1].astype(_i32), pcol]).reshape(NBLKP, K)
    val2 = jnp.concatenate([adj_values, jnp.zeros((padn,), _f32)]).reshape(NBLKP, K)
    ego3 = _prep_ego3(user_embedding_weight, item_embedding_weight)
    norms = []
    for (wg, bg, wb, bb) in ((W_gcn_0, b_gcn_0, W_bi_0, b_bi_0),
                             (W_gcn_1, b_gcn_1, W_bi_1, b_bi_1),
                             (W_gcn_2, b_gcn_2, W_bi_2, b_bi_2)):
        side3 = _sc_spmm(row2, col2, val2, ego3)
        ego3, nrm = _tc_dense(side3, ego3, wg, bg, wb, bb)
        norms.append(nrm)
    return _assemble(user_embedding_weight, item_embedding_weight, *norms)


# speedup vs baseline: 1.1783x; 1.0188x over previous
import functools

import jax
import jax.numpy as jnp
from jax import lax
from jax.experimental import pallas as pl
from jax.experimental.pallas import tpu as pltpu
from jax.experimental.pallas import tpu_sc as plsc

N_TOTAL = 50000
E = 800000
DIM = 64
HALF = 32
K = 128
IDXB = 16
NBUF = 4
LOOK = 2
CPS = 400
NITER = CPS // IDXB
NBLKP = 16 * CPS
EP = NBLKP * K
NSUB = 16
RSUB = 3128
RLAST = N_TOTAL - 15 * RSUB

_f32 = jnp.float32
_i32 = jnp.int32
_DNUMS = lax.GatherDimensionNumbers(
    offset_dims=(), collapsed_slice_dims=(0,), start_index_map=(0,))


def _bcast_lane(vec16, jj):
    return lax.gather(vec16, jnp.full((16, 1), jj, _i32), _DNUMS,
                      slice_sizes=(1,),
                      mode=lax.GatherScatterMode.PROMISE_IN_BOUNDS)


def _sc_spmm(rc2, val2, ego3):
    mesh = plsc.VectorSubcoreMesh(core_axis_name="c", subcore_axis_name="s")

    @functools.partial(
        pl.kernel,
        out_type=jax.ShapeDtypeStruct((2, N_TOTAL, HALF), _f32),
        mesh=mesh,
        scratch_types=[
            pltpu.VMEM((2, IDXB, K), _i32),
            pltpu.VMEM((2, IDXB, K), _i32),
            pltpu.VMEM((2, IDXB, K), _f32),
            [pltpu.VMEM((K, HALF), _f32)] * NBUF,
            pltpu.VMEM_SHARED((N_TOTAL, HALF), _f32),
            [pltpu.SemaphoreType.DMA] * NBUF,
            [pltpu.SemaphoreType.DMA] * NBUF,
            pltpu.SemaphoreType.DMA,
        ],
        compiler_params=pltpu.CompilerParams(use_tc_tiling_on_sc=False),
    )
    def spmm(rc_h, val_h, ego_h, side_h,
             rowv, colv, valv, bufs, side_sh, gsems, ssems, isem):
        cid = lax.axis_index("c")
        sid = lax.axis_index("s")

        zero16 = jnp.zeros((16,), _f32)

        def zrow(r, carry):
            bufs[0][r, pl.ds(0, 16)] = zero16
            bufs[0][r, pl.ds(16, 16)] = zero16
            return carry

        lax.fori_loop(0, K, zrow, 0)

        def zcopy(i, carry):
            pltpu.sync_copy(
                bufs[0], side_sh.at[pl.ds(sid * RSUB + i * K, K)])
            return carry

        lax.fori_loop(0, 24, zcopy, 0)

        @pl.when(sid < 15)
        def _():
            pltpu.sync_copy(bufs[0].at[pl.ds(0, 56)],
                            side_sh.at[pl.ds(sid * RSUB + 24 * K, 56)])

        @pl.when(sid == 15)
        def _():
            pltpu.sync_copy(bufs[0].at[pl.ds(0, 8)],
                            side_sh.at[pl.ds(15 * RSUB + 24 * K, 8)])

        plsc.subcore_barrier()

        def scale(buf, valrow):
            def g_body(g, carry):
                val16 = valrow[pl.ds(g * 16, 16)]
                for jj in range(16):
                    r = g * 16 + jj
                    bv = _bcast_lane(val16, jj)
                    buf[r, pl.ds(0, 16)] = buf[r, pl.ds(0, 16)] * bv
                    buf[r, pl.ds(16, 16)] = buf[r, pl.ds(16, 16)] * bv
                return carry

            lax.fori_loop(0, K // 16, g_body, 0)

        ego_c = ego_h.at[cid]
        base0 = sid * CPS

        pltpu.sync_copy(rc_h.at[1].at[pl.ds(base0, IDXB)], colv.at[0])
        pltpu.sync_copy(rc_h.at[0].at[pl.ds(base0, IDXB)], rowv.at[0])
        pltpu.sync_copy(val_h.at[pl.ds(base0, IDXB)], valv.at[0])

        def one_iter(t, carry):
            tp = lax.rem(t, 2)
            tn = 1 - tp
            nbase = base0 + (t + 1) * IDXB

            @pl.when(t < NITER - 1)
            def _():
                pltpu.async_copy(rc_h.at[1].at[pl.ds(nbase, IDXB)], colv.at[tn], isem)
                pltpu.async_copy(rc_h.at[0].at[pl.ds(nbase, IDXB)], rowv.at[tn], isem)
                pltpu.async_copy(val_h.at[pl.ds(nbase, IDXB)], valv.at[tn], isem)

            cv = colv.at[tp]
            rv = rowv.at[tp]
            gd = [None] * NBUF
            sd = [None] * NBUF
            for s in range(LOOK):
                gd[s] = pltpu.async_copy(ego_c.at[cv.at[s]], bufs[s], gsems[s])
            for s in range(IDXB):
                b = s % NBUF
                if s + LOOK < IDXB:
                    nb = (s + LOOK) % NBUF
                    if sd[nb] is not None:
                        sd[nb].wait()
                    gd[nb] = pltpu.async_copy(
                        ego_c.at[cv.at[s + LOOK]], bufs[nb], gsems[nb])
                gd[b].wait()
                scale(bufs[b], valv.at[tp].at[s])
                sd[b] = pltpu.async_copy(
                    bufs[b], side_sh.at[rv.at[s]], ssems[b], add=True)
            for b in range(NBUF):
                sd[b].wait()

            @pl.when(t < NITER - 1)
            def _():
                pltpu.make_async_copy(rc_h.at[1].at[pl.ds(nbase, IDXB)],
                                      colv.at[tn], isem).wait()
                pltpu.make_async_copy(rc_h.at[0].at[pl.ds(nbase, IDXB)],
                                      rowv.at[tn], isem).wait()
                pltpu.make_async_copy(val_h.at[pl.ds(nbase, IDXB)],
                                      valv.at[tn], isem).wait()

            return carry

        lax.fori_loop(0, NITER, one_iter, 0)
        plsc.subcore_barrier()

        sl = pl.ds(sid * RSUB, RSUB)
        sl_last = pl.ds(15 * RSUB, RLAST)

        @pl.when(sid < 15)
        def _():
            pltpu.sync_copy(side_sh.at[sl], side_h.at[cid].at[sl])

        @pl.when(sid == 15)
        def _():
            pltpu.sync_copy(side_sh.at[sl_last], side_h.at[cid].at[sl_last])

    return spmm(rc2, val2, ego3)


BLK = 5000


def _tc_body(s3_ref, e3_ref, wg_ref, bg_ref, wb_ref, bb_ref,
             onext_ref, onorm_ref):
    s = jnp.concatenate([s3_ref[0], s3_ref[1]], axis=1)
    eg = jnp.concatenate([e3_ref[0], e3_ref[1]], axis=1)
    sum_emb = jnp.dot(s, wg_ref[...], preferred_element_type=_f32) + bg_ref[...]
    bi_emb = (jnp.dot(eg * s, wb_ref[...], preferred_element_type=_f32)
              + bb_ref[...])
    e = sum_emb + bi_emb
    e = jnp.where(e >= 0, e, 0.2 * e)
    nr = jnp.sqrt(jnp.sum(e * e, axis=1, keepdims=True))
    d = e / jnp.maximum(nr, 1e-12)
    onext_ref[0] = e[:, :HALF]
    onext_ref[1] = e[:, HALF:]
    onorm_ref[...] = d


def _tc_dense(side3, ego3, w_gcn, b_gcn, w_bi, b_bi):
    grid = (N_TOTAL // BLK,)
    pair_spec = pl.BlockSpec((2, BLK, HALF), lambda i: (0, i, 0))
    w_spec = pl.BlockSpec((DIM, DIM), lambda i: (0, 0))
    b_spec = pl.BlockSpec((1, DIM), lambda i: (0, 0))
    return pl.pallas_call(
        _tc_body,
        grid=grid,
        in_specs=[pair_spec, pair_spec, w_spec, b_spec, w_spec, b_spec],
        out_specs=[pair_spec, pl.BlockSpec((BLK, DIM), lambda i: (i, 0))],
        out_shape=[
            jax.ShapeDtypeStruct((2, N_TOTAL, HALF), _f32),
            jax.ShapeDtypeStruct((N_TOTAL, DIM), _f32),
        ],
    )(side3, ego3, w_gcn, b_gcn, w_bi, b_bi)


PBLK = 5000


def _prep_body(u_ref, it_ref, out_ref):
    i = pl.program_id(0)

    @pl.when(i < 5)
    def _():
        e = u_ref[...]
        out_ref[0] = e[:, :HALF]
        out_ref[1] = e[:, HALF:]

    @pl.when(i >= 5)
    def _():
        e = it_ref[...]
        out_ref[0] = e[:, :HALF]
        out_ref[1] = e[:, HALF:]


def _prep_ego3(user_w, item_w):
    spec = pl.BlockSpec((PBLK, DIM), lambda i: (jnp.minimum(i, 4), 0))
    spec_it = pl.BlockSpec((PBLK, DIM), lambda i: (jnp.maximum(i - 5, 0), 0))
    return pl.pallas_call(
        _prep_body,
        grid=(10,),
        in_specs=[spec, spec_it],
        out_specs=pl.BlockSpec((2, PBLK, HALF), lambda i: (0, i, 0)),
        out_shape=jax.ShapeDtypeStruct((2, N_TOTAL, HALF), _f32),
    )(user_w, item_w)


def _asm_body(u_ref, it_ref, n1u_ref, n1i_ref, n2u_ref, n2i_ref,
              n3u_ref, n3i_ref, us_ref, is_ref):
    us_ref[...] = jnp.concatenate(
        [u_ref[...], n1u_ref[...], n2u_ref[...], n3u_ref[...]], axis=1)
    is_ref[...] = jnp.concatenate(
        [it_ref[...], n1i_ref[...], n2i_ref[...], n3i_ref[...]], axis=1)


def _assemble(user_w, item_w, n1, n2, n3):
    nu = N_TOTAL // 2
    spec_u = pl.BlockSpec((PBLK, DIM), lambda i: (i, 0))
    spec_i = pl.BlockSpec((PBLK, DIM), lambda i: (i + 5, 0))
    out_spec = pl.BlockSpec((PBLK, 4 * DIM), lambda i: (i, 0))
    return pl.pallas_call(
        _asm_body,
        grid=(5,),
        in_specs=[spec_u, spec_u, spec_u, spec_i, spec_u, spec_i,
                  spec_u, spec_i],
        out_specs=[out_spec, out_spec],
        out_shape=[jax.ShapeDtypeStruct((nu, 4 * DIM), _f32),
                   jax.ShapeDtypeStruct((nu, 4 * DIM), _f32)],
    )(user_w, item_w, n1, n1, n2, n2, n3, n3)


def kernel(adj_indices, adj_values, user_embedding_weight, item_embedding_weight,
           W_gcn_0, b_gcn_0, W_bi_0, b_bi_0,
           W_gcn_1, b_gcn_1, W_bi_1, b_bi_1,
           W_gcn_2, b_gcn_2, W_bi_2, b_bi_2):
    padn = EP - E
    ar = jnp.arange(padn, dtype=_i32)
    prc = jnp.stack([ar % N_TOTAL, (ar * 61) % N_TOTAL])
    rc2 = jnp.concatenate([adj_indices.astype(_i32), prc],
                          axis=1).reshape(2, NBLKP, K)
    val2 = jnp.concatenate([adj_values, jnp.zeros((padn,), _f32)]).reshape(NBLKP, K)
    ego3 = _prep_ego3(user_embedding_weight, item_embedding_weight)
    norms = []
    for (wg, bg, wb, bb) in ((W_gcn_0, b_gcn_0, W_bi_0, b_bi_0),
                             (W_gcn_1, b_gcn_1, W_bi_1, b_bi_1),
                             (W_gcn_2, b_gcn_2, W_bi_2, b_bi_2)):
        side3 = _sc_spmm(rc2, val2, ego3)
        ego3, nrm = _tc_dense(side3, ego3, wg, bg, wb, bb)
        norms.append(nrm)
    return _assemble(user_embedding_weight, item_embedding_weight, *norms)


# speedup vs baseline: 1.2401x; 1.0524x over previous
import functools

import jax
import jax.numpy as jnp
from jax import lax
from jax.experimental import pallas as pl
from jax.experimental.pallas import tpu as pltpu
from jax.experimental.pallas import tpu_sc as plsc

N_TOTAL = 50000
E = 800000
DIM = 64
HALF = 32
K = 128
IDXB = 16
NBUF = 4
LOOK = 2
CPS = 400
NITER = CPS // IDXB
NBLKP = 16 * CPS
EP = NBLKP * K
NSUB = 16
RSUB = 3128
RLAST = N_TOTAL - 15 * RSUB

_f32 = jnp.float32
_i32 = jnp.int32
_DNUMS = lax.GatherDimensionNumbers(
    offset_dims=(), collapsed_slice_dims=(0,), start_index_map=(0,))


def _bcast_lane(vec16, jj):
    return lax.gather(vec16, jnp.full((16, 1), jj, _i32), _DNUMS,
                      slice_sizes=(1,),
                      mode=lax.GatherScatterMode.PROMISE_IN_BOUNDS)


def _sc_spmm(rc2, val2, ego3):
    mesh = plsc.VectorSubcoreMesh(core_axis_name="c", subcore_axis_name="s")

    @functools.partial(
        pl.kernel,
        out_type=jax.ShapeDtypeStruct((2, N_TOTAL, HALF), _f32),
        mesh=mesh,
        scratch_types=[
            pltpu.VMEM((2, IDXB, K), _i32),
            pltpu.VMEM((2, IDXB, K), _i32),
            pltpu.VMEM((2, IDXB, K), _f32),
            [pltpu.VMEM((K, HALF), _f32)] * NBUF,
            pltpu.VMEM_SHARED((N_TOTAL, HALF), _f32),
            [pltpu.SemaphoreType.DMA] * NBUF,
            [pltpu.SemaphoreType.DMA] * NBUF,
            pltpu.SemaphoreType.DMA,
        ],
        compiler_params=pltpu.CompilerParams(use_tc_tiling_on_sc=False),
    )
    def spmm(rc_h, val_h, ego_h, side_h,
             rowv, colv, valv, bufs, side_sh, gsems, ssems, isem):
        cid = lax.axis_index("c")
        sid = lax.axis_index("s")

        zero16 = jnp.zeros((16,), _f32)

        def zrow(r, carry):
            bufs[0][r, pl.ds(0, 16)] = zero16
            bufs[0][r, pl.ds(16, 16)] = zero16
            return carry

        lax.fori_loop(0, K, zrow, 0)

        def zcopy(i, carry):
            pltpu.sync_copy(
                bufs[0], side_sh.at[pl.ds(sid * RSUB + i * K, K)])
            return carry

        lax.fori_loop(0, 24, zcopy, 0)

        @pl.when(sid < 15)
        def _():
            pltpu.sync_copy(bufs[0].at[pl.ds(0, 56)],
                            side_sh.at[pl.ds(sid * RSUB + 24 * K, 56)])

        @pl.when(sid == 15)
        def _():
            pltpu.sync_copy(bufs[0].at[pl.ds(0, 8)],
                            side_sh.at[pl.ds(15 * RSUB + 24 * K, 8)])

        plsc.subcore_barrier()

        def scale(buf, valrow):
            def g_body(g, carry):
                val16 = valrow[pl.ds(g * 16, 16)]
                for jj in range(16):
                    r = g * 16 + jj
                    bv = _bcast_lane(val16, jj)
                    buf[r, pl.ds(0, 16)] = buf[r, pl.ds(0, 16)] * bv
                    buf[r, pl.ds(16, 16)] = buf[r, pl.ds(16, 16)] * bv
                return carry

            lax.fori_loop(0, K // 16, g_body, 0)

        ego_c = ego_h.at[cid]
        base0 = sid * CPS

        pltpu.sync_copy(rc_h.at[1].at[pl.ds(base0, IDXB)], colv.at[0])
        pltpu.sync_copy(rc_h.at[0].at[pl.ds(base0, IDXB)], rowv.at[0])
        pltpu.sync_copy(val_h.at[pl.ds(base0, IDXB)], valv.at[0])
        for s in range(LOOK):
            pltpu.async_copy(ego_c.at[colv.at[0].at[s]], bufs[s], gsems[s])

        def one_iter(t, carry):
            tp = lax.rem(t, 2)
            tn = 1 - tp
            nbase = base0 + (t + 1) * IDXB

            @pl.when(t < NITER - 1)
            def _():
                pltpu.async_copy(rc_h.at[1].at[pl.ds(nbase, IDXB)], colv.at[tn], isem)
                pltpu.async_copy(rc_h.at[0].at[pl.ds(nbase, IDXB)], rowv.at[tn], isem)
                pltpu.async_copy(val_h.at[pl.ds(nbase, IDXB)], valv.at[tn], isem)

            cv = colv.at[tp]
            rv = rowv.at[tp]
            cvn = colv.at[tn]
            gd = [None] * NBUF
            sd = [None] * NBUF
            for s in range(IDXB):
                b = s % NBUF
                nb = (s + LOOK) % NBUF
                if sd[nb] is not None:
                    sd[nb].wait()
                    sd[nb] = None
                if s + LOOK < IDXB:
                    gd[nb] = pltpu.async_copy(
                        ego_c.at[cv.at[s + LOOK]], bufs[nb], gsems[nb])
                else:
                    if s == IDXB - LOOK:
                        @pl.when(t < NITER - 1)
                        def _():
                            pltpu.make_async_copy(
                                rc_h.at[1].at[pl.ds(nbase, IDXB)],
                                colv.at[tn], isem).wait()
                            pltpu.make_async_copy(
                                rc_h.at[0].at[pl.ds(nbase, IDXB)],
                                rowv.at[tn], isem).wait()
                            pltpu.make_async_copy(
                                val_h.at[pl.ds(nbase, IDXB)],
                                valv.at[tn], isem).wait()

                    nxt = s + LOOK - IDXB

                    @pl.when(t < NITER - 1)
                    def _(nxt=nxt, nb=nb):
                        pltpu.async_copy(
                            ego_c.at[cvn.at[nxt]], bufs[nb], gsems[nb])
                if gd[b] is not None:
                    gd[b].wait()
                else:
                    pltpu.make_async_copy(
                        ego_c.at[cv.at[s]], bufs[b], gsems[b]).wait()
                scale(bufs[b], valv.at[tp].at[s])
                sd[b] = pltpu.async_copy(
                    bufs[b], side_sh.at[rv.at[s]], ssems[b], add=True)
            for b in range(NBUF):
                if sd[b] is not None:
                    sd[b].wait()

            return carry

        lax.fori_loop(0, NITER, one_iter, 0)
        plsc.subcore_barrier()

        sl = pl.ds(sid * RSUB, RSUB)
        sl_last = pl.ds(15 * RSUB, RLAST)

        @pl.when(sid < 15)
        def _():
            pltpu.sync_copy(side_sh.at[sl], side_h.at[cid].at[sl])

        @pl.when(sid == 15)
        def _():
            pltpu.sync_copy(side_sh.at[sl_last], side_h.at[cid].at[sl_last])

    return spmm(rc2, val2, ego3)


BLK = 5000


def _tc_body(s3_ref, e3_ref, wg_ref, bg_ref, wb_ref, bb_ref,
             onext_ref, onorm_ref):
    s = jnp.concatenate([s3_ref[0], s3_ref[1]], axis=1)
    eg = jnp.concatenate([e3_ref[0], e3_ref[1]], axis=1)
    sum_emb = jnp.dot(s, wg_ref[...], preferred_element_type=_f32) + bg_ref[...]
    bi_emb = (jnp.dot(eg * s, wb_ref[...], preferred_element_type=_f32)
              + bb_ref[...])
    e = sum_emb + bi_emb
    e = jnp.where(e >= 0, e, 0.2 * e)
    nr = jnp.sqrt(jnp.sum(e * e, axis=1, keepdims=True))
    d = e / jnp.maximum(nr, 1e-12)
    onext_ref[0] = e[:, :HALF]
    onext_ref[1] = e[:, HALF:]
    onorm_ref[...] = d


def _tc_dense(side3, ego3, w_gcn, b_gcn, w_bi, b_bi):
    grid = (N_TOTAL // BLK,)
    pair_spec = pl.BlockSpec((2, BLK, HALF), lambda i: (0, i, 0))
    w_spec = pl.BlockSpec((DIM, DIM), lambda i: (0, 0))
    b_spec = pl.BlockSpec((1, DIM), lambda i: (0, 0))
    return pl.pallas_call(
        _tc_body,
        grid=grid,
        in_specs=[pair_spec, pair_spec, w_spec, b_spec, w_spec, b_spec],
        out_specs=[pair_spec, pl.BlockSpec((BLK, DIM), lambda i: (i, 0))],
        out_shape=[
            jax.ShapeDtypeStruct((2, N_TOTAL, HALF), _f32),
            jax.ShapeDtypeStruct((N_TOTAL, DIM), _f32),
        ],
    )(side3, ego3, w_gcn, b_gcn, w_bi, b_bi)


PBLK = 5000


def _prep_body(u_ref, it_ref, out_ref):
    i = pl.program_id(0)

    @pl.when(i < 5)
    def _():
        e = u_ref[...]
        out_ref[0] = e[:, :HALF]
        out_ref[1] = e[:, HALF:]

    @pl.when(i >= 5)
    def _():
        e = it_ref[...]
        out_ref[0] = e[:, :HALF]
        out_ref[1] = e[:, HALF:]


def _prep_ego3(user_w, item_w):
    spec = pl.BlockSpec((PBLK, DIM), lambda i: (jnp.minimum(i, 4), 0))
    spec_it = pl.BlockSpec((PBLK, DIM), lambda i: (jnp.maximum(i - 5, 0), 0))
    return pl.pallas_call(
        _prep_body,
        grid=(10,),
        in_specs=[spec, spec_it],
        out_specs=pl.BlockSpec((2, PBLK, HALF), lambda i: (0, i, 0)),
        out_shape=jax.ShapeDtypeStruct((2, N_TOTAL, HALF), _f32),
    )(user_w, item_w)


def _asm_body(u_ref, it_ref, n1u_ref, n1i_ref, n2u_ref, n2i_ref,
              n3u_ref, n3i_ref, us_ref, is_ref):
    us_ref[...] = jnp.concatenate(
        [u_ref[...], n1u_ref[...], n2u_ref[...], n3u_ref[...]], axis=1)
    is_ref[...] = jnp.concatenate(
        [it_ref[...], n1i_ref[...], n2i_ref[...], n3i_ref[...]], axis=1)


def _assemble(user_w, item_w, n1, n2, n3):
    nu = N_TOTAL // 2
    spec_u = pl.BlockSpec((PBLK, DIM), lambda i: (i, 0))
    spec_i = pl.BlockSpec((PBLK, DIM), lambda i: (i + 5, 0))
    out_spec = pl.BlockSpec((PBLK, 4 * DIM), lambda i: (i, 0))
    return pl.pallas_call(
        _asm_body,
        grid=(5,),
        in_specs=[spec_u, spec_u, spec_u, spec_i, spec_u, spec_i,
                  spec_u, spec_i],
        out_specs=[out_spec, out_spec],
        out_shape=[jax.ShapeDtypeStruct((nu, 4 * DIM), _f32),
                   jax.ShapeDtypeStruct((nu, 4 * DIM), _f32)],
    )(user_w, item_w, n1, n1, n2, n2, n3, n3)


def kernel(adj_indices, adj_values, user_embedding_weight, item_embedding_weight,
           W_gcn_0, b_gcn_0, W_bi_0, b_bi_0,
           W_gcn_1, b_gcn_1, W_bi_1, b_bi_1,
           W_gcn_2, b_gcn_2, W_bi_2, b_bi_2):
    padn = EP - E
    ar = jnp.arange(padn, dtype=_i32)
    prc = jnp.stack([ar % N_TOTAL, (ar * 61) % N_TOTAL])
    rc2 = jnp.concatenate([adj_indices.astype(_i32), prc],
                          axis=1).reshape(2, NBLKP, K)
    val2 = jnp.concatenate([adj_values, jnp.zeros((padn,), _f32)]).reshape(NBLKP, K)
    ego3 = _prep_ego3(user_embedding_weight, item_embedding_weight)
    norms = []
    for (wg, bg, wb, bb) in ((W_gcn_0, b_gcn_0, W_bi_0, b_bi_0),
                             (W_gcn_1, b_gcn_1, W_bi_1, b_bi_1),
                             (W_gcn_2, b_gcn_2, W_bi_2, b_bi_2)):
        side3 = _sc_spmm(rc2, val2, ego3)
        ego3, nrm = _tc_dense(side3, ego3, wg, bg, wb, bb)
        norms.append(nrm)
    return _assemble(user_embedding_weight, item_embedding_weight, *norms)
